# baseline (speedup 1.0000x reference)
import functools

import jax
import jax.numpy as jnp
from jax import lax
from jax.experimental import pallas as pl
from jax.experimental.pallas import tpu as pltpu
from jax.experimental.pallas import tpu_sc as plsc

N = 50000
E = 800000
D = 32

NC = 2
NS = 16
NW = NC * NS
CHUNK = 128
NCHUNKS = E // CHUNK
BASE_CH = NCHUNKS // NW
EXTRA = NCHUNKS % NW
ROWS_PER_TILE = N // NS
KB = 8
NB = BASE_CH // KB
KBS = 4
NBS = BASE_CH // KBS

BN = 10000
GN = N // BN
E4 = E // 4
BE4 = 4000
GE4 = E4 // BE4
BEP = 4000
GEP = E4 // BEP

_F32 = jnp.float32


def _full(spec_shape):
    return pl.BlockSpec(spec_shape, lambda i: (0,) * len(spec_shape))


def _rows(bshape):
    return pl.BlockSpec(bshape, lambda i: (i,) + (0,) * (len(bshape) - 1))


def _node_encoder(x, w1, b1, w2, b2, w3, b3):
    def body(x_ref, w1r, b1r, w2r, b2r, w3r, b3r, o_ref):
        h = jnp.maximum(x_ref[...] * w1r[...] + b1r[...], 0.0)
        h = jnp.maximum(jnp.dot(h, w2r[...], preferred_element_type=_F32) + b2r[...], 0.0)
        h = jnp.maximum(jnp.dot(h, w2r[...], preferred_element_type=_F32) + b2r[...], 0.0)
        h = jnp.maximum(jnp.dot(h, w3r[...], preferred_element_type=_F32) + b3r[...], 0.0)
        o_ref[...] = h

    return pl.pallas_call(
        body,
        grid=(GN,),
        in_specs=[_rows((BN, 1)), _full((1, D)), _full((1, D)), _full((D, D)),
                  _full((1, D)), _full((D, D)), _full((1, D))],
        out_specs=_rows((BN, D)),
        out_shape=jax.ShapeDtypeStruct((N, D), _F32),
    )(x, w1, b1, w2, b2, w3, b3)


def _edge_encoder(ea_t, gi, w1, b1, w2, b2, w3, b3, wa, wb):
    def body(a_ref, gi_ref, w1r, b1r, w2r, b2r, w3r, b3r, war, wbr, o_ref, mo_ref):
        a = a_ref[...]
        h = jax.lax.dot_general(a, w1r[...], (((0,), (0,)), ((), ())),
                                preferred_element_type=_F32)
        h = jnp.maximum(h + b1r[...], 0.0)
        h = jnp.maximum(jnp.dot(h, w2r[...], preferred_element_type=_F32) + b2r[...], 0.0)
        h = jnp.maximum(jnp.dot(h, w2r[...], preferred_element_type=_F32) + b2r[...], 0.0)
        h = jnp.maximum(jnp.dot(h, w3r[...], preferred_element_type=_F32) + b3r[...], 0.0)
        h4 = h.reshape(BEP, 4, D)
        hp = jnp.concatenate([h4[:, k, :] for k in range(4)], axis=1)
        o_ref[...] = hp
        m = (jnp.dot(gi_ref[...], war[...], preferred_element_type=_F32)
             + jnp.dot(hp, wbr[...], preferred_element_type=_F32))
        mo_ref[...] = jnp.maximum(m, 0.0)

    return pl.pallas_call(
        body,
        grid=(GEP,),
        in_specs=[pl.BlockSpec((3, 4 * BEP), lambda i: (0, i)), _rows((BEP, 128)),
                  _full((3, D)), _full((1, D)), _full((D, D)), _full((1, D)),
                  _full((D, D)), _full((1, D)), _full((128, 128)),
                  _full((128, 128))],
        out_specs=[_rows((BEP, 128)), _rows((BEP, 128))],
        out_shape=[jax.ShapeDtypeStruct((E4, 128), _F32),
                   jax.ShapeDtypeStruct((E4, 128), _F32)],
    )(ea_t, gi, w1, b1, w2, b2, w3, b3, wa, wb)


def _message(gi, he, wa, wb):
    def body(g_ref, h_ref, wa_r, wb_r, o_ref):
        m = (jnp.dot(g_ref[...], wa_r[...], preferred_element_type=_F32)
             + jnp.dot(h_ref[...], wb_r[...], preferred_element_type=_F32))
        o_ref[...] = jnp.maximum(m, 0.0)

    return pl.pallas_call(
        body,
        grid=(GE4,),
        in_specs=[_rows((BE4, 128)), _rows((BE4, 128)),
                  _full((128, 128)), _full((128, 128))],
        out_specs=_rows((BE4, 128)),
        out_shape=jax.ShapeDtypeStruct((E4, 128), _F32),
    )(gi, he, wa, wb)


def _fused_edge(he, gi, gj, w0, w1, w2, b, wm0, wm1):
    def body(h_ref, gi_ref, gj_ref, w0r, w1r, w2r, br, wm0r, wm1r, ho_ref, mo_ref):
        h = (jnp.dot(h_ref[...], w0r[...], preferred_element_type=_F32)
             + jnp.dot(gi_ref[...], w1r[...], preferred_element_type=_F32)
             + jnp.dot(gj_ref[...], w2r[...], preferred_element_type=_F32)
             + br[...])
        h = jnp.maximum(h, 0.0)
        ho_ref[...] = h
        m = (jnp.dot(gi_ref[...], wm0r[...], preferred_element_type=_F32)
             + jnp.dot(h, wm1r[...], preferred_element_type=_F32))
        mo_ref[...] = jnp.maximum(m, 0.0)

    return pl.pallas_call(
        body,
        grid=(GE4,),
        in_specs=[_rows((BE4, 128)), _rows((BE4, 128)), _rows((BE4, 128)),
                  _full((128, 128)), _full((128, 128)), _full((128, 128)),
                  _full((1, 128)), _full((128, 128)), _full((128, 128))],
        out_specs=[_rows((BE4, 128)), _rows((BE4, 128))],
        out_shape=[jax.ShapeDtypeStruct((E4, 128), _F32),
                   jax.ShapeDtypeStruct((E4, 128), _F32)],
    )(he, gi, gj, w0, w1, w2, b, wm0, wm1)


def _final_edge(ea_t, he, gi, gj, w0, w1, w2, b):
    def body(ea_ref, h_ref, gi_ref, gj_ref, w0r, w1r, w2r, br, o_ref):
        res = (jnp.dot(h_ref[...], w0r[...], preferred_element_type=_F32)
               + jnp.dot(gi_ref[...], w1r[...], preferred_element_type=_F32)
               + jnp.dot(gj_ref[...], w2r[...], preferred_element_type=_F32)
               + br[...])
        at = jnp.transpose(ea_ref[...])
        a4 = at.reshape(BEP, 4, 3)
        eap = jnp.concatenate([a4[:, k, :] for k in range(4)], axis=1)
        o_ref[...] = res + eap

    return pl.pallas_call(
        body,
        grid=(GEP,),
        in_specs=[pl.BlockSpec((3, 4 * BEP), lambda i: (0, i)), _rows((BEP, 128)),
                  _rows((BEP, 128)), _rows((BEP, 128)), _full((128, 12)),
                  _full((128, 12)), _full((128, 12)), _full((1, 12))],
        out_specs=_rows((BEP, 12)),
        out_shape=jax.ShapeDtypeStruct((E4, 12), _F32),
    )(ea_t, he, gi, gj, w0, w1, w2, b)


def _inv_counts(cnt):
    def body(c_ref, o_ref):
        c = c_ref[0, :, 0:1] + c_ref[1, :, 0:1]
        o_ref[...] = 1.0 / jnp.maximum(c, 1.0)

    return pl.pallas_call(
        body,
        grid=(GN,),
        in_specs=[pl.BlockSpec((NC, BN, 16), lambda i: (0, i, 0))],
        out_specs=_rows((BN, 1)),
        out_shape=jax.ShapeDtypeStruct((N, 1), _F32),
    )(cnt)


def _node_finalize(acc, inv):
    def body(a_ref, inv_ref, o_ref):
        o_ref[...] = (a_ref[0] + a_ref[1]) * inv_ref[...]

    return pl.pallas_call(
        body,
        grid=(GN,),
        in_specs=[pl.BlockSpec((NC, BN, D), lambda i: (0, i, 0)), _rows((BN, 1))],
        out_specs=_rows((BN, D)),
        out_shape=jax.ShapeDtypeStruct((N, D), _F32),
    )(acc, inv)


@functools.cache
def _mesh():
    return plsc.VectorSubcoreMesh(core_axis_name="c", subcore_axis_name="s",
                                  num_cores=NC, num_subcores=NS)


def _worker_range(wid):
    c0 = jnp.where(wid < EXTRA, wid * (BASE_CH + 1),
                   EXTRA * (BASE_CH + 1) + (wid - EXTRA) * BASE_CH)
    nch = BASE_CH + jnp.where(wid < EXTRA, 1, 0)
    return c0, nch


def _sc_gather2(table, idx_i, idx_j):
    @functools.partial(
        pl.kernel,
        out_type=(jax.ShapeDtypeStruct((E, D), _F32),
                  jax.ShapeDtypeStruct((E, D), _F32)),
        mesh=_mesh(),
        compiler_params=pltpu.CompilerParams(use_tc_tiling_on_sc=False),
        scratch_types=[
            pltpu.VMEM((KB, CHUNK), jnp.int32), pltpu.VMEM((KB * CHUNK, D), _F32),
            pltpu.VMEM((KB, CHUNK), jnp.int32), pltpu.VMEM((KB * CHUNK, D), _F32),
            pltpu.SemaphoreType.DMA,
        ],
    )
    def k(table_h, ii_h, jj_h, oi_h, oj_h, iv1, rv1, iv2, rv2, sem):
        c = lax.axis_index("c")
        s = lax.axis_index("s")
        wid = s * NC + c
        c0, nch = _worker_range(wid)

        def batch(b, carry):
            cb = c0 + b * KB
            pltpu.sync_copy(ii_h.at[pl.ds(cb, KB)], iv1)
            pltpu.sync_copy(jj_h.at[pl.ds(cb, KB)], iv2)
            cps = []
            for kk in range(KB):
                cps.append(pltpu.async_copy(
                    table_h.at[iv1.at[kk]],
                    rv1.at[pl.ds(kk * CHUNK, CHUNK)], sem))
                cps.append(pltpu.async_copy(
                    table_h.at[iv2.at[kk]],
                    rv2.at[pl.ds(kk * CHUNK, CHUNK)], sem))
            for cp in cps:
                cp.wait()
            pltpu.sync_copy(rv1, oi_h.at[pl.ds(cb * CHUNK, KB * CHUNK)])
            pltpu.sync_copy(rv2, oj_h.at[pl.ds(cb * CHUNK, KB * CHUNK)])
            return carry

        lax.fori_loop(0, NB, batch, 0)

        def tailb(t, carry):
            r = c0 + NB * KB + t
            pltpu.sync_copy(ii_h.at[r], iv1.at[0])
            pltpu.sync_copy(jj_h.at[r], iv2.at[0])
            cp1 = pltpu.async_copy(table_h.at[iv1.at[0]],
                                   rv1.at[pl.ds(0, CHUNK)], sem)
            cp2 = pltpu.async_copy(table_h.at[iv2.at[0]],
                                   rv2.at[pl.ds(0, CHUNK)], sem)
            cp1.wait()
            cp2.wait()
            pltpu.sync_copy(rv1.at[pl.ds(0, CHUNK)],
                            oi_h.at[pl.ds(r * CHUNK, CHUNK)])
            pltpu.sync_copy(rv2.at[pl.ds(0, CHUNK)],
                            oj_h.at[pl.ds(r * CHUNK, CHUNK)])
            return carry

        lax.fori_loop(0, nch - NB * KB, tailb, 0)

    return k(table, idx_i, idx_j)


def _sc_gather1(table, idx_i):
    @functools.partial(
        pl.kernel,
        out_type=jax.ShapeDtypeStruct((E, D), _F32),
        mesh=_mesh(),
        compiler_params=pltpu.CompilerParams(use_tc_tiling_on_sc=False),
        scratch_types=[
            pltpu.VMEM((KB, CHUNK), jnp.int32), pltpu.VMEM((KB * CHUNK, D), _F32),
            pltpu.SemaphoreType.DMA,
        ],
    )
    def k(table_h, ii_h, oi_h, iv1, rv1, sem):
        c = lax.axis_index("c")
        s = lax.axis_index("s")
        wid = s * NC + c
        c0, nch = _worker_range(wid)

        def batch(b, carry):
            cb = c0 + b * KB
            pltpu.sync_copy(ii_h.at[pl.ds(cb, KB)], iv1)
            cps = [pltpu.async_copy(table_h.at[iv1.at[kk]],
                                    rv1.at[pl.ds(kk * CHUNK, CHUNK)], sem)
                   for kk in range(KB)]
            for cp in cps:
                cp.wait()
            pltpu.sync_copy(rv1, oi_h.at[pl.ds(cb * CHUNK, KB * CHUNK)])
            return carry

        lax.fori_loop(0, NB, batch, 0)

        def tailb(t, carry):
            r = c0 + NB * KB + t
            pltpu.sync_copy(ii_h.at[r], iv1.at[0])
            pltpu.async_copy(table_h.at[iv1.at[0]],
                             rv1.at[pl.ds(0, CHUNK)], sem).wait()
            pltpu.sync_copy(rv1.at[pl.ds(0, CHUNK)],
                            oi_h.at[pl.ds(r * CHUNK, CHUNK)])
            return carry

        lax.fori_loop(0, nch - NB * KB, tailb, 0)

    return k(table, idx_i)


def _sc_scatter_add(vals, idx, zeros):
    @functools.partial(
        pl.kernel,
        out_type=jax.ShapeDtypeStruct((NC, N, D), _F32),
        mesh=_mesh(),
        compiler_params=pltpu.CompilerParams(use_tc_tiling_on_sc=False),
        scratch_types=[
            pltpu.VMEM_SHARED((N, D), _F32),
            pltpu.VMEM((KBS, CHUNK), jnp.int32), pltpu.VMEM((KBS * CHUNK, D), _F32),
            pltpu.SemaphoreType.DMA,
        ],
    )
    def k(vals_h, idx_h, z_h, out_h, acc_sh, iv, rv, sem):
        c = lax.axis_index("c")
        s = lax.axis_index("s")
        wid = s * NC + c
        row0 = s * ROWS_PER_TILE
        pltpu.sync_copy(z_h, acc_sh.at[pl.ds(row0, ROWS_PER_TILE)])
        plsc.subcore_barrier()
        c0, nch = _worker_range(wid)

        def batch(b, carry):
            cb = c0 + b * KBS
            pltpu.sync_copy(idx_h.at[pl.ds(cb, KBS)], iv)
            pltpu.sync_copy(vals_h.at[pl.ds(cb * CHUNK, KBS * CHUNK)], rv)
            cps = [pltpu.async_copy(rv.at[pl.ds(kk * CHUNK, CHUNK)],
                                    acc_sh.at[iv.at[kk]], sem, add=True)
                   for kk in range(KBS)]
            for cp in cps:
                cp.wait()
            return carry

        lax.fori_loop(0, NBS, batch, 0)

        def tailb(t, carry):
            r = c0 + NBS * KBS + t
            pltpu.sync_copy(idx_h.at[r], iv.at[0])
            pltpu.sync_copy(vals_h.at[pl.ds(r * CHUNK, CHUNK)],
                            rv.at[pl.ds(0, CHUNK)])
            pltpu.sync_copy(rv.at[pl.ds(0, CHUNK)], acc_sh.at[iv.at[0]], add=True)
            return carry

        lax.fori_loop(0, nch - NBS * KBS, tailb, 0)
        plsc.subcore_barrier()
        pltpu.sync_copy(acc_sh.at[pl.ds(row0, ROWS_PER_TILE)],
                        out_h.at[c, pl.ds(row0, ROWS_PER_TILE)])

    return k(vals, idx, zeros)


def _sc_count(idx, zeros16, ones16):
    @functools.partial(
        pl.kernel,
        out_type=jax.ShapeDtypeStruct((NC, N, 16), _F32),
        mesh=_mesh(),
        compiler_params=pltpu.CompilerParams(use_tc_tiling_on_sc=False),
        scratch_types=[
            pltpu.VMEM_SHARED((N, 16), _F32),
            pltpu.VMEM((KB, CHUNK), jnp.int32), pltpu.VMEM((CHUNK, 16), _F32),
            pltpu.SemaphoreType.DMA,
        ],
    )
    def k(idx_h, z_h, ones_h, out_h, acc_sh, iv, ov, sem):
        c = lax.axis_index("c")
        s = lax.axis_index("s")
        wid = s * NC + c
        row0 = s * ROWS_PER_TILE
        pltpu.sync_copy(z_h, acc_sh.at[pl.ds(row0, ROWS_PER_TILE)])
        pltpu.sync_copy(ones_h, ov)
        plsc.subcore_barrier()
        c0, nch = _worker_range(wid)

        def batch(b, carry):
            cb = c0 + b * KB
            pltpu.sync_copy(idx_h.at[pl.ds(cb, KB)], iv)
            cps = [pltpu.async_copy(ov, acc_sh.at[iv.at[kk]], sem, add=True)
                   for kk in range(KB)]
            for cp in cps:
                cp.wait()
            return carry

        lax.fori_loop(0, NB, batch, 0)

        def tailb(t, carry):
            r = c0 + NB * KB + t
            pltpu.sync_copy(idx_h.at[r], iv.at[0])
            pltpu.sync_copy(ov, acc_sh.at[iv.at[0]], add=True)
            return carry

        lax.fori_loop(0, nch - NB * KB, tailb, 0)
        plsc.subcore_barrier()
        pltpu.sync_copy(acc_sh.at[pl.ds(row0, ROWS_PER_TILE)],
                        out_h.at[c, pl.ds(row0, ROWS_PER_TILE)])

    return k(idx, zeros16, ones16)


def _bd4(w):
    return jax.scipy.linalg.block_diag(w, w, w, w)


def _bt4(bvec):
    return jnp.tile(bvec.reshape(1, -1), (1, 4))


def kernel(x, edge_attr, edge_index, W_s_enc_node, b_s_enc_node, W_enc_node,
           b_enc_node, W_e_enc_node, b_e_enc_node, W_s_enc_edge, b_s_enc_edge,
           W_enc_edge, b_enc_edge, W_e_enc_edge, b_e_enc_edge, W_s_node,
           W_node, W_e_node, W_s_edge, b_s_edge, W_edge, b_edge, W_e_edge,
           b_e_edge):
    idx_i = edge_index[0].reshape(NCHUNKS, CHUNK)
    idx_j = edge_index[1].reshape(NCHUNKS, CHUNK)
    ea_t = jnp.transpose(edge_attr)

    zeros_d = jnp.zeros((ROWS_PER_TILE, D), _F32)
    zeros16 = jnp.zeros((ROWS_PER_TILE, 16), _F32)
    ones16 = jnp.ones((CHUNK, 16), _F32)

    r2 = lambda v: v.reshape(1, -1)

    h_node = _node_encoder(x, W_s_enc_node, r2(b_s_enc_node), W_enc_node,
                           r2(b_enc_node), W_e_enc_node, r2(b_e_enc_node))
    cnt = _sc_count(idx_j, zeros16, ones16)
    inv = _inv_counts(cnt)

    gi = _sc_gather1(h_node, idx_i).reshape(E4, 128)
    h_edge, m = _edge_encoder(ea_t, gi, W_s_enc_edge, r2(b_s_enc_edge),
                              W_enc_edge, r2(b_enc_edge),
                              W_e_enc_edge, r2(b_e_enc_edge),
                              _bd4(W_s_node[:D]), _bd4(W_s_node[D:]))
    acc = _sc_scatter_add(m.reshape(E, D), idx_j, zeros_d)
    h_node = _node_finalize(acc, inv)

    for w_eu, b_eu, w_msg in ((W_s_edge, b_s_edge, W_node),
                              (W_edge, b_edge, W_node),
                              (W_edge, b_edge, W_e_node)):
        gi, gj = _sc_gather2(h_node, idx_i, idx_j)
        gi = gi.reshape(E4, 128)
        gj = gj.reshape(E4, 128)
        h_edge, m = _fused_edge(h_edge, gi, gj, _bd4(w_eu[:D]),
                                _bd4(w_eu[D:2 * D]), _bd4(w_eu[2 * D:]),
                                _bt4(b_eu), _bd4(w_msg[:D]), _bd4(w_msg[D:]))
        acc = _sc_scatter_add(m.reshape(E, D), idx_j, zeros_d)
        h_node = _node_finalize(acc, inv)

    gi, gj = _sc_gather2(h_node, idx_i, idx_j)
    gi = gi.reshape(E4, 128)
    gj = gj.reshape(E4, 128)
    res = _final_edge(ea_t, h_edge, gi, gj, _bd4(W_e_edge[:D]),
                      _bd4(W_e_edge[D:2 * D]), _bd4(W_e_edge[2 * D:]),
                      _bt4(b_e_edge))
    return res.reshape(E, 3)


# speedup vs baseline: 1.0557x; 1.0557x over previous
import functools

import jax
import jax.numpy as jnp
from jax import lax
from jax.experimental import pallas as pl
from jax.experimental.pallas import tpu as pltpu
from jax.experimental.pallas import tpu_sc as plsc

N = 50000
E = 800000
D = 32

NC = 2
NS = 16
NW = NC * NS
CHUNK = 128
NCHUNKS = E // CHUNK
BASE_CH = NCHUNKS // NW
EXTRA = NCHUNKS % NW
ROWS_PER_TILE = N // NS
KB = 8
NB = BASE_CH // KB
KBS = 4
NBS = BASE_CH // KBS

BN = 10000
GN = N // BN
E4 = E // 4
BE4 = 4000
GE4 = E4 // BE4
BEP = 4000
GEP = E4 // BEP

_F32 = jnp.float32


def _full(spec_shape):
    return pl.BlockSpec(spec_shape, lambda i: (0,) * len(spec_shape))


def _rows(bshape):
    return pl.BlockSpec(bshape, lambda i: (i,) + (0,) * (len(bshape) - 1))


def _node_encoder(x, w1, b1, w2, b2, w3, b3):
    def body(x_ref, w1r, b1r, w2r, b2r, w3r, b3r, o_ref):
        h = jnp.maximum(x_ref[...] * w1r[...] + b1r[...], 0.0)
        h = jnp.maximum(jnp.dot(h, w2r[...], preferred_element_type=_F32) + b2r[...], 0.0)
        h = jnp.maximum(jnp.dot(h, w2r[...], preferred_element_type=_F32) + b2r[...], 0.0)
        h = jnp.maximum(jnp.dot(h, w3r[...], preferred_element_type=_F32) + b3r[...], 0.0)
        o_ref[...] = h

    return pl.pallas_call(
        body,
        grid=(GN,),
        in_specs=[_rows((BN, 1)), _full((1, D)), _full((1, D)), _full((D, D)),
                  _full((1, D)), _full((D, D)), _full((1, D))],
        out_specs=_rows((BN, D)),
        out_shape=jax.ShapeDtypeStruct((N, D), _F32),
    )(x, w1, b1, w2, b2, w3, b3)


def _edge_encoder(ea_t, gi, w1, b1, w2, b2, w3, b3, wa, wb):
    def body(a_ref, gi_ref, w1r, b1r, w2r, b2r, w3r, b3r, war, wbr, o_ref, mo_ref):
        a = a_ref[...]
        h = jax.lax.dot_general(a, w1r[...], (((0,), (0,)), ((), ())),
                                preferred_element_type=_F32)
        h = jnp.maximum(h + b1r[...], 0.0)
        h = jnp.maximum(jnp.dot(h, w2r[...], preferred_element_type=_F32) + b2r[...], 0.0)
        h = jnp.maximum(jnp.dot(h, w2r[...], preferred_element_type=_F32) + b2r[...], 0.0)
        h = jnp.maximum(jnp.dot(h, w3r[...], preferred_element_type=_F32) + b3r[...], 0.0)
        h4 = h.reshape(BEP, 4, D)
        hp = jnp.concatenate([h4[:, k, :] for k in range(4)], axis=1)
        o_ref[...] = hp
        m = (jnp.dot(gi_ref[...], war[...], preferred_element_type=_F32)
             + jnp.dot(hp, wbr[...], preferred_element_type=_F32))
        mo_ref[...] = jnp.maximum(m, 0.0)

    return pl.pallas_call(
        body,
        grid=(GEP,),
        in_specs=[pl.BlockSpec((3, 4 * BEP), lambda i: (0, i)), _rows((BEP, 128)),
                  _full((3, D)), _full((1, D)), _full((D, D)), _full((1, D)),
                  _full((D, D)), _full((1, D)), _full((128, 128)),
                  _full((128, 128))],
        out_specs=[_rows((BEP, 128)), _rows((BEP, 128))],
        out_shape=[jax.ShapeDtypeStruct((E4, 128), _F32),
                   jax.ShapeDtypeStruct((E4, 128), _F32)],
    )(ea_t, gi, w1, b1, w2, b2, w3, b3, wa, wb)


def _message(gi, he, wa, wb):
    def body(g_ref, h_ref, wa_r, wb_r, o_ref):
        m = (jnp.dot(g_ref[...], wa_r[...], preferred_element_type=_F32)
             + jnp.dot(h_ref[...], wb_r[...], preferred_element_type=_F32))
        o_ref[...] = jnp.maximum(m, 0.0)

    return pl.pallas_call(
        body,
        grid=(GE4,),
        in_specs=[_rows((BE4, 128)), _rows((BE4, 128)),
                  _full((128, 128)), _full((128, 128))],
        out_specs=_rows((BE4, 128)),
        out_shape=jax.ShapeDtypeStruct((E4, 128), _F32),
    )(gi, he, wa, wb)


def _fused_edge(he, gi, gj, w0, w1, w2, b, wm0, wm1):
    def body(h_ref, gi_ref, gj_ref, w0r, w1r, w2r, br, wm0r, wm1r, ho_ref, mo_ref):
        h = (jnp.dot(h_ref[...], w0r[...], preferred_element_type=_F32)
             + jnp.dot(gi_ref[...], w1r[...], preferred_element_type=_F32)
             + jnp.dot(gj_ref[...], w2r[...], preferred_element_type=_F32)
             + br[...])
        h = jnp.maximum(h, 0.0)
        ho_ref[...] = h
        m = (jnp.dot(gi_ref[...], wm0r[...], preferred_element_type=_F32)
             + jnp.dot(h, wm1r[...], preferred_element_type=_F32))
        mo_ref[...] = jnp.maximum(m, 0.0)

    return pl.pallas_call(
        body,
        grid=(GE4,),
        in_specs=[_rows((BE4, 128)), _rows((BE4, 128)), _rows((BE4, 128)),
                  _full((128, 128)), _full((128, 128)), _full((128, 128)),
                  _full((1, 128)), _full((128, 128)), _full((128, 128))],
        out_specs=[_rows((BE4, 128)), _rows((BE4, 128))],
        out_shape=[jax.ShapeDtypeStruct((E4, 128), _F32),
                   jax.ShapeDtypeStruct((E4, 128), _F32)],
    )(he, gi, gj, w0, w1, w2, b, wm0, wm1)


def _final_edge(he, gi, gj, w0, w1, w2, b):
    def body(h_ref, gi_ref, gj_ref, w0r, w1r, w2r, br, o_ref):
        o_ref[...] = (jnp.dot(h_ref[...], w0r[...], preferred_element_type=_F32)
                      + jnp.dot(gi_ref[...], w1r[...], preferred_element_type=_F32)
                      + jnp.dot(gj_ref[...], w2r[...], preferred_element_type=_F32)
                      + br[...])

    return pl.pallas_call(
        body,
        grid=(GE4,),
        in_specs=[_rows((BE4, 128)), _rows((BE4, 128)), _rows((BE4, 128)),
                  _full((128, 12)), _full((128, 12)), _full((128, 12)),
                  _full((1, 12))],
        out_specs=_rows((BE4, 12)),
        out_shape=jax.ShapeDtypeStruct((E4, 12), _F32),
    )(he, gi, gj, w0, w1, w2, b)


def _inv_counts(cnt):
    def body(c_ref, o_ref):
        c = c_ref[0, :, 0:1] + c_ref[1, :, 0:1]
        o_ref[...] = 1.0 / jnp.maximum(c, 1.0)

    return pl.pallas_call(
        body,
        grid=(GN,),
        in_specs=[pl.BlockSpec((NC, BN, 16), lambda i: (0, i, 0))],
        out_specs=_rows((BN, 1)),
        out_shape=jax.ShapeDtypeStruct((N, 1), _F32),
    )(cnt)


def _node_finalize(acc, inv):
    def body(a_ref, inv_ref, o_ref):
        o_ref[...] = (a_ref[0] + a_ref[1]) * inv_ref[...]

    return pl.pallas_call(
        body,
        grid=(GN,),
        in_specs=[pl.BlockSpec((NC, BN, D), lambda i: (0, i, 0)), _rows((BN, 1))],
        out_specs=_rows((BN, D)),
        out_shape=jax.ShapeDtypeStruct((N, D), _F32),
    )(acc, inv)


@functools.cache
def _mesh():
    return plsc.VectorSubcoreMesh(core_axis_name="c", subcore_axis_name="s",
                                  num_cores=NC, num_subcores=NS)


def _worker_range(wid):
    c0 = jnp.where(wid < EXTRA, wid * (BASE_CH + 1),
                   EXTRA * (BASE_CH + 1) + (wid - EXTRA) * BASE_CH)
    nch = BASE_CH + jnp.where(wid < EXTRA, 1, 0)
    return c0, nch


def _sc_gather2(table, idx_i, idx_j):
    @functools.partial(
        pl.kernel,
        out_type=(jax.ShapeDtypeStruct((E, D), _F32),
                  jax.ShapeDtypeStruct((E, D), _F32)),
        mesh=_mesh(),
        compiler_params=pltpu.CompilerParams(use_tc_tiling_on_sc=False),
        scratch_types=[
            pltpu.VMEM((KB, CHUNK), jnp.int32), pltpu.VMEM((KB * CHUNK, D), _F32),
            pltpu.VMEM((KB, CHUNK), jnp.int32), pltpu.VMEM((KB * CHUNK, D), _F32),
            pltpu.SemaphoreType.DMA,
        ],
    )
    def k(table_h, ii_h, jj_h, oi_h, oj_h, iv1, rv1, iv2, rv2, sem):
        c = lax.axis_index("c")
        s = lax.axis_index("s")
        wid = s * NC + c
        c0, nch = _worker_range(wid)

        def batch(b, carry):
            cb = c0 + b * KB
            pltpu.sync_copy(ii_h.at[pl.ds(cb, KB)], iv1)
            pltpu.sync_copy(jj_h.at[pl.ds(cb, KB)], iv2)
            cps = []
            for kk in range(KB):
                cps.append(pltpu.async_copy(
                    table_h.at[iv1.at[kk]],
                    rv1.at[pl.ds(kk * CHUNK, CHUNK)], sem))
                cps.append(pltpu.async_copy(
                    table_h.at[iv2.at[kk]],
                    rv2.at[pl.ds(kk * CHUNK, CHUNK)], sem))
            for cp in cps:
                cp.wait()
            pltpu.sync_copy(rv1, oi_h.at[pl.ds(cb * CHUNK, KB * CHUNK)])
            pltpu.sync_copy(rv2, oj_h.at[pl.ds(cb * CHUNK, KB * CHUNK)])
            return carry

        lax.fori_loop(0, NB, batch, 0)

        def tailb(t, carry):
            r = c0 + NB * KB + t
            pltpu.sync_copy(ii_h.at[r], iv1.at[0])
            pltpu.sync_copy(jj_h.at[r], iv2.at[0])
            cp1 = pltpu.async_copy(table_h.at[iv1.at[0]],
                                   rv1.at[pl.ds(0, CHUNK)], sem)
            cp2 = pltpu.async_copy(table_h.at[iv2.at[0]],
                                   rv2.at[pl.ds(0, CHUNK)], sem)
            cp1.wait()
            cp2.wait()
            pltpu.sync_copy(rv1.at[pl.ds(0, CHUNK)],
                            oi_h.at[pl.ds(r * CHUNK, CHUNK)])
            pltpu.sync_copy(rv2.at[pl.ds(0, CHUNK)],
                            oj_h.at[pl.ds(r * CHUNK, CHUNK)])
            return carry

        lax.fori_loop(0, nch - NB * KB, tailb, 0)

    return k(table, idx_i, idx_j)


def _sc_gather1(table, idx_i):
    @functools.partial(
        pl.kernel,
        out_type=jax.ShapeDtypeStruct((E, D), _F32),
        mesh=_mesh(),
        compiler_params=pltpu.CompilerParams(use_tc_tiling_on_sc=False),
        scratch_types=[
            pltpu.VMEM((KB, CHUNK), jnp.int32), pltpu.VMEM((KB * CHUNK, D), _F32),
            pltpu.SemaphoreType.DMA,
        ],
    )
    def k(table_h, ii_h, oi_h, iv1, rv1, sem):
        c = lax.axis_index("c")
        s = lax.axis_index("s")
        wid = s * NC + c
        c0, nch = _worker_range(wid)

        def batch(b, carry):
            cb = c0 + b * KB
            pltpu.sync_copy(ii_h.at[pl.ds(cb, KB)], iv1)
            cps = [pltpu.async_copy(table_h.at[iv1.at[kk]],
                                    rv1.at[pl.ds(kk * CHUNK, CHUNK)], sem)
                   for kk in range(KB)]
            for cp in cps:
                cp.wait()
            pltpu.sync_copy(rv1, oi_h.at[pl.ds(cb * CHUNK, KB * CHUNK)])
            return carry

        lax.fori_loop(0, NB, batch, 0)

        def tailb(t, carry):
            r = c0 + NB * KB + t
            pltpu.sync_copy(ii_h.at[r], iv1.at[0])
            pltpu.async_copy(table_h.at[iv1.at[0]],
                             rv1.at[pl.ds(0, CHUNK)], sem).wait()
            pltpu.sync_copy(rv1.at[pl.ds(0, CHUNK)],
                            oi_h.at[pl.ds(r * CHUNK, CHUNK)])
            return carry

        lax.fori_loop(0, nch - NB * KB, tailb, 0)

    return k(table, idx_i)


def _sc_scatter_add(vals, idx, zeros):
    @functools.partial(
        pl.kernel,
        out_type=jax.ShapeDtypeStruct((NC, N, D), _F32),
        mesh=_mesh(),
        compiler_params=pltpu.CompilerParams(use_tc_tiling_on_sc=False),
        scratch_types=[
            pltpu.VMEM_SHARED((N, D), _F32),
            pltpu.VMEM((KBS, CHUNK), jnp.int32), pltpu.VMEM((KBS * CHUNK, D), _F32),
            pltpu.SemaphoreType.DMA,
        ],
    )
    def k(vals_h, idx_h, z_h, out_h, acc_sh, iv, rv, sem):
        c = lax.axis_index("c")
        s = lax.axis_index("s")
        wid = s * NC + c
        row0 = s * ROWS_PER_TILE
        pltpu.sync_copy(z_h, acc_sh.at[pl.ds(row0, ROWS_PER_TILE)])
        plsc.subcore_barrier()
        c0, nch = _worker_range(wid)

        def batch(b, carry):
            cb = c0 + b * KBS
            pltpu.sync_copy(idx_h.at[pl.ds(cb, KBS)], iv)
            pltpu.sync_copy(vals_h.at[pl.ds(cb * CHUNK, KBS * CHUNK)], rv)
            cps = [pltpu.async_copy(rv.at[pl.ds(kk * CHUNK, CHUNK)],
                                    acc_sh.at[iv.at[kk]], sem, add=True)
                   for kk in range(KBS)]
            for cp in cps:
                cp.wait()
            return carry

        lax.fori_loop(0, NBS, batch, 0)

        def tailb(t, carry):
            r = c0 + NBS * KBS + t
            pltpu.sync_copy(idx_h.at[r], iv.at[0])
            pltpu.sync_copy(vals_h.at[pl.ds(r * CHUNK, CHUNK)],
                            rv.at[pl.ds(0, CHUNK)])
            pltpu.sync_copy(rv.at[pl.ds(0, CHUNK)], acc_sh.at[iv.at[0]], add=True)
            return carry

        lax.fori_loop(0, nch - NBS * KBS, tailb, 0)
        plsc.subcore_barrier()
        pltpu.sync_copy(acc_sh.at[pl.ds(row0, ROWS_PER_TILE)],
                        out_h.at[c, pl.ds(row0, ROWS_PER_TILE)])

    return k(vals, idx, zeros)


def _sc_count(idx, zeros16, ones16):
    @functools.partial(
        pl.kernel,
        out_type=jax.ShapeDtypeStruct((NC, N, 16), _F32),
        mesh=_mesh(),
        compiler_params=pltpu.CompilerParams(use_tc_tiling_on_sc=False),
        scratch_types=[
            pltpu.VMEM_SHARED((N, 16), _F32),
            pltpu.VMEM((KB, CHUNK), jnp.int32), pltpu.VMEM((CHUNK, 16), _F32),
            pltpu.SemaphoreType.DMA,
        ],
    )
    def k(idx_h, z_h, ones_h, out_h, acc_sh, iv, ov, sem):
        c = lax.axis_index("c")
        s = lax.axis_index("s")
        wid = s * NC + c
        row0 = s * ROWS_PER_TILE
        pltpu.sync_copy(z_h, acc_sh.at[pl.ds(row0, ROWS_PER_TILE)])
        pltpu.sync_copy(ones_h, ov)
        plsc.subcore_barrier()
        c0, nch = _worker_range(wid)

        def batch(b, carry):
            cb = c0 + b * KB
            pltpu.sync_copy(idx_h.at[pl.ds(cb, KB)], iv)
            cps = [pltpu.async_copy(ov, acc_sh.at[iv.at[kk]], sem, add=True)
                   for kk in range(KB)]
            for cp in cps:
                cp.wait()
            return carry

        lax.fori_loop(0, NB, batch, 0)

        def tailb(t, carry):
            r = c0 + NB * KB + t
            pltpu.sync_copy(idx_h.at[r], iv.at[0])
            pltpu.sync_copy(ov, acc_sh.at[iv.at[0]], add=True)
            return carry

        lax.fori_loop(0, nch - NB * KB, tailb, 0)
        plsc.subcore_barrier()
        pltpu.sync_copy(acc_sh.at[pl.ds(row0, ROWS_PER_TILE)],
                        out_h.at[c, pl.ds(row0, ROWS_PER_TILE)])

    return k(idx, zeros16, ones16)


def _bd4(w):
    return jax.scipy.linalg.block_diag(w, w, w, w)


def _bt4(bvec):
    return jnp.tile(bvec.reshape(1, -1), (1, 4))


def kernel(x, edge_attr, edge_index, W_s_enc_node, b_s_enc_node, W_enc_node,
           b_enc_node, W_e_enc_node, b_e_enc_node, W_s_enc_edge, b_s_enc_edge,
           W_enc_edge, b_enc_edge, W_e_enc_edge, b_e_enc_edge, W_s_node,
           W_node, W_e_node, W_s_edge, b_s_edge, W_edge, b_edge, W_e_edge,
           b_e_edge):
    idx_i = edge_index[0].reshape(NCHUNKS, CHUNK)
    idx_j = edge_index[1].reshape(NCHUNKS, CHUNK)
    ea_t = jnp.transpose(edge_attr)

    zeros_d = jnp.zeros((ROWS_PER_TILE, D), _F32)
    zeros16 = jnp.zeros((ROWS_PER_TILE, 16), _F32)
    ones16 = jnp.ones((CHUNK, 16), _F32)

    r2 = lambda v: v.reshape(1, -1)

    h_node = _node_encoder(x, W_s_enc_node, r2(b_s_enc_node), W_enc_node,
                           r2(b_enc_node), W_e_enc_node, r2(b_e_enc_node))
    cnt = _sc_count(idx_j, zeros16, ones16)
    inv = _inv_counts(cnt)

    gi = _sc_gather1(h_node, idx_i).reshape(E4, 128)
    h_edge, m = _edge_encoder(ea_t, gi, W_s_enc_edge, r2(b_s_enc_edge),
                              W_enc_edge, r2(b_enc_edge),
                              W_e_enc_edge, r2(b_e_enc_edge),
                              _bd4(W_s_node[:D]), _bd4(W_s_node[D:]))
    acc = _sc_scatter_add(m.reshape(E, D), idx_j, zeros_d)
    h_node = _node_finalize(acc, inv)

    for w_eu, b_eu, w_msg in ((W_s_edge, b_s_edge, W_node),
                              (W_edge, b_edge, W_node),
                              (W_edge, b_edge, W_e_node)):
        gi, gj = _sc_gather2(h_node, idx_i, idx_j)
        gi = gi.reshape(E4, 128)
        gj = gj.reshape(E4, 128)
        h_edge, m = _fused_edge(h_edge, gi, gj, _bd4(w_eu[:D]),
                                _bd4(w_eu[D:2 * D]), _bd4(w_eu[2 * D:]),
                                _bt4(b_eu), _bd4(w_msg[:D]), _bd4(w_msg[D:]))
        acc = _sc_scatter_add(m.reshape(E, D), idx_j, zeros_d)
        h_node = _node_finalize(acc, inv)

    gi, gj = _sc_gather2(h_node, idx_i, idx_j)
    gi = gi.reshape(E4, 128)
    gj = gj.reshape(E4, 128)
    res = _final_edge(h_edge, gi, gj, _bd4(W_e_edge[:D]),
                      _bd4(W_e_edge[D:2 * D]), _bd4(W_e_edge[2 * D:]),
                      _bt4(b_e_edge))
    return edge_attr + res.reshape(E, 3)


# speedup vs baseline: 1.1007x; 1.0426x over previous
import functools

import jax
import jax.numpy as jnp
from jax import lax
from jax.experimental import pallas as pl
from jax.experimental.pallas import tpu as pltpu
from jax.experimental.pallas import tpu_sc as plsc

N = 50000
E = 800000
D = 32

NC = 2
NS = 16
NW = NC * NS
CHUNK = 128
NCHUNKS = E // CHUNK
BASE_CH = NCHUNKS // NW
EXTRA = NCHUNKS % NW
ROWS_PER_TILE = N // NS
KB = 12
NB = BASE_CH // KB
KBS = 6
NBS = BASE_CH // KBS

BN = 10000
GN = N // BN
E4 = E // 4
BE4 = 4000
GE4 = E4 // BE4
BEP = 4000
GEP = E4 // BEP

_F32 = jnp.float32


def _full(spec_shape):
    return pl.BlockSpec(spec_shape, lambda i: (0,) * len(spec_shape))


def _rows(bshape):
    return pl.BlockSpec(bshape, lambda i: (i,) + (0,) * (len(bshape) - 1))


def _node_encoder(x, w1, b1, w2, b2, w3, b3):
    def body(x_ref, w1r, b1r, w2r, b2r, w3r, b3r, o_ref):
        h = jnp.maximum(x_ref[...] * w1r[...] + b1r[...], 0.0)
        h = jnp.maximum(jnp.dot(h, w2r[...], preferred_element_type=_F32) + b2r[...], 0.0)
        h = jnp.maximum(jnp.dot(h, w2r[...], preferred_element_type=_F32) + b2r[...], 0.0)
        h = jnp.maximum(jnp.dot(h, w3r[...], preferred_element_type=_F32) + b3r[...], 0.0)
        o_ref[...] = h

    return pl.pallas_call(
        body,
        grid=(GN,),
        in_specs=[_rows((BN, 1)), _full((1, D)), _full((1, D)), _full((D, D)),
                  _full((1, D)), _full((D, D)), _full((1, D))],
        out_specs=_rows((BN, D)),
        out_shape=jax.ShapeDtypeStruct((N, D), _F32),
    )(x, w1, b1, w2, b2, w3, b3)


def _edge_encoder(ea_t, gi, w1, b1, w2, b2, w3, b3, wa, wb):
    def body(a_ref, gi_ref, w1r, b1r, w2r, b2r, w3r, b3r, war, wbr, o_ref, mo_ref):
        a = a_ref[...]
        h = jax.lax.dot_general(a, w1r[...], (((0,), (0,)), ((), ())),
                                preferred_element_type=_F32)
        h = jnp.maximum(h + b1r[...], 0.0)
        h = jnp.maximum(jnp.dot(h, w2r[...], preferred_element_type=_F32) + b2r[...], 0.0)
        h = jnp.maximum(jnp.dot(h, w2r[...], preferred_element_type=_F32) + b2r[...], 0.0)
        h = jnp.maximum(jnp.dot(h, w3r[...], preferred_element_type=_F32) + b3r[...], 0.0)
        h4 = h.reshape(BEP, 4, D)
        hp = jnp.concatenate([h4[:, k, :] for k in range(4)], axis=1)
        o_ref[...] = hp
        m = (jnp.dot(gi_ref[...], war[...], preferred_element_type=_F32)
             + jnp.dot(hp, wbr[...], preferred_element_type=_F32))
        mo_ref[...] = jnp.maximum(m, 0.0)

    return pl.pallas_call(
        body,
        grid=(GEP,),
        in_specs=[pl.BlockSpec((3, 4 * BEP), lambda i: (0, i)), _rows((BEP, 128)),
                  _full((3, D)), _full((1, D)), _full((D, D)), _full((1, D)),
                  _full((D, D)), _full((1, D)), _full((128, 128)),
                  _full((128, 128))],
        out_specs=[_rows((BEP, 128)), _rows((BEP, 128))],
        out_shape=[jax.ShapeDtypeStruct((E4, 128), _F32),
                   jax.ShapeDtypeStruct((E4, 128), _F32)],
    )(ea_t, gi, w1, b1, w2, b2, w3, b3, wa, wb)


def _message(gi, he, wa, wb):
    def body(g_ref, h_ref, wa_r, wb_r, o_ref):
        m = (jnp.dot(g_ref[...], wa_r[...], preferred_element_type=_F32)
             + jnp.dot(h_ref[...], wb_r[...], preferred_element_type=_F32))
        o_ref[...] = jnp.maximum(m, 0.0)

    return pl.pallas_call(
        body,
        grid=(GE4,),
        in_specs=[_rows((BE4, 128)), _rows((BE4, 128)),
                  _full((128, 128)), _full((128, 128))],
        out_specs=_rows((BE4, 128)),
        out_shape=jax.ShapeDtypeStruct((E4, 128), _F32),
    )(gi, he, wa, wb)


def _fused_edge(he, gi, gj, w0, w1, w2, b, wm0, wm1):
    def body(h_ref, gi_ref, gj_ref, w0r, w1r, w2r, br, wm0r, wm1r, ho_ref, mo_ref):
        h = (jnp.dot(h_ref[...], w0r[...], preferred_element_type=_F32)
             + jnp.dot(gi_ref[...], w1r[...], preferred_element_type=_F32)
             + jnp.dot(gj_ref[...], w2r[...], preferred_element_type=_F32)
             + br[...])
        h = jnp.maximum(h, 0.0)
        ho_ref[...] = h
        m = (jnp.dot(gi_ref[...], wm0r[...], preferred_element_type=_F32)
             + jnp.dot(h, wm1r[...], preferred_element_type=_F32))
        mo_ref[...] = jnp.maximum(m, 0.0)

    return pl.pallas_call(
        body,
        grid=(GE4,),
        in_specs=[_rows((BE4, 128)), _rows((BE4, 128)), _rows((BE4, 128)),
                  _full((128, 128)), _full((128, 128)), _full((128, 128)),
                  _full((1, 128)), _full((128, 128)), _full((128, 128))],
        out_specs=[_rows((BE4, 128)), _rows((BE4, 128))],
        out_shape=[jax.ShapeDtypeStruct((E4, 128), _F32),
                   jax.ShapeDtypeStruct((E4, 128), _F32)],
    )(he, gi, gj, w0, w1, w2, b, wm0, wm1)


def _final_edge(he, gi, gj, w0, w1, w2, b):
    def body(h_ref, gi_ref, gj_ref, w0r, w1r, w2r, br, o_ref):
        o_ref[...] = (jnp.dot(h_ref[...], w0r[...], preferred_element_type=_F32)
                      + jnp.dot(gi_ref[...], w1r[...], preferred_element_type=_F32)
                      + jnp.dot(gj_ref[...], w2r[...], preferred_element_type=_F32)
                      + br[...])

    return pl.pallas_call(
        body,
        grid=(GE4,),
        in_specs=[_rows((BE4, 128)), _rows((BE4, 128)), _rows((BE4, 128)),
                  _full((128, 12)), _full((128, 12)), _full((128, 12)),
                  _full((1, 12))],
        out_specs=_rows((BE4, 12)),
        out_shape=jax.ShapeDtypeStruct((E4, 12), _F32),
    )(he, gi, gj, w0, w1, w2, b)


def _inv_counts(cnt):
    def body(c_ref, o_ref):
        c = c_ref[0, :, 0:1] + c_ref[1, :, 0:1]
        o_ref[...] = 1.0 / jnp.maximum(c, 1.0)

    return pl.pallas_call(
        body,
        grid=(GN,),
        in_specs=[pl.BlockSpec((NC, BN, 16), lambda i: (0, i, 0))],
        out_specs=_rows((BN, 1)),
        out_shape=jax.ShapeDtypeStruct((N, 1), _F32),
    )(cnt)


def _node_finalize(acc, inv):
    def body(a_ref, inv_ref, o_ref):
        o_ref[...] = (a_ref[0] + a_ref[1]) * inv_ref[...]

    return pl.pallas_call(
        body,
        grid=(GN,),
        in_specs=[pl.BlockSpec((NC, BN, D), lambda i: (0, i, 0)), _rows((BN, 1))],
        out_specs=_rows((BN, D)),
        out_shape=jax.ShapeDtypeStruct((N, D), _F32),
    )(acc, inv)


@functools.cache
def _mesh():
    return plsc.VectorSubcoreMesh(core_axis_name="c", subcore_axis_name="s",
                                  num_cores=NC, num_subcores=NS)


def _worker_range(wid):
    c0 = jnp.where(wid < EXTRA, wid * (BASE_CH + 1),
                   EXTRA * (BASE_CH + 1) + (wid - EXTRA) * BASE_CH)
    nch = BASE_CH + jnp.where(wid < EXTRA, 1, 0)
    return c0, nch


def _sc_gather2(table, idx_i, idx_j):
    @functools.partial(
        pl.kernel,
        out_type=(jax.ShapeDtypeStruct((E, D), _F32),
                  jax.ShapeDtypeStruct((E, D), _F32)),
        mesh=_mesh(),
        compiler_params=pltpu.CompilerParams(use_tc_tiling_on_sc=False),
        scratch_types=[
            pltpu.VMEM((KB, CHUNK), jnp.int32), pltpu.VMEM((KB * CHUNK, D), _F32),
            pltpu.VMEM((KB, CHUNK), jnp.int32), pltpu.VMEM((KB * CHUNK, D), _F32),
            pltpu.SemaphoreType.DMA,
        ],
    )
    def k(table_h, ii_h, jj_h, oi_h, oj_h, iv1, rv1, iv2, rv2, sem):
        c = lax.axis_index("c")
        s = lax.axis_index("s")
        wid = s * NC + c
        c0, nch = _worker_range(wid)

        def batch(b, carry):
            cb = c0 + b * KB
            pltpu.sync_copy(ii_h.at[pl.ds(cb, KB)], iv1)
            pltpu.sync_copy(jj_h.at[pl.ds(cb, KB)], iv2)
            cps = []
            for kk in range(KB):
                cps.append(pltpu.async_copy(
                    table_h.at[iv1.at[kk]],
                    rv1.at[pl.ds(kk * CHUNK, CHUNK)], sem))
                cps.append(pltpu.async_copy(
                    table_h.at[iv2.at[kk]],
                    rv2.at[pl.ds(kk * CHUNK, CHUNK)], sem))
            for cp in cps:
                cp.wait()
            pltpu.sync_copy(rv1, oi_h.at[pl.ds(cb * CHUNK, KB * CHUNK)])
            pltpu.sync_copy(rv2, oj_h.at[pl.ds(cb * CHUNK, KB * CHUNK)])
            return carry

        lax.fori_loop(0, NB, batch, 0)

        def tailb(t, carry):
            r = c0 + NB * KB + t
            pltpu.sync_copy(ii_h.at[r], iv1.at[0])
            pltpu.sync_copy(jj_h.at[r], iv2.at[0])
            cp1 = pltpu.async_copy(table_h.at[iv1.at[0]],
                                   rv1.at[pl.ds(0, CHUNK)], sem)
            cp2 = pltpu.async_copy(table_h.at[iv2.at[0]],
                                   rv2.at[pl.ds(0, CHUNK)], sem)
            cp1.wait()
            cp2.wait()
            pltpu.sync_copy(rv1.at[pl.ds(0, CHUNK)],
                            oi_h.at[pl.ds(r * CHUNK, CHUNK)])
            pltpu.sync_copy(rv2.at[pl.ds(0, CHUNK)],
                            oj_h.at[pl.ds(r * CHUNK, CHUNK)])
            return carry

        lax.fori_loop(0, nch - NB * KB, tailb, 0)

    return k(table, idx_i, idx_j)


def _sc_gather1(table, idx_i):
    @functools.partial(
        pl.kernel,
        out_type=jax.ShapeDtypeStruct((E, D), _F32),
        mesh=_mesh(),
        compiler_params=pltpu.CompilerParams(use_tc_tiling_on_sc=False),
        scratch_types=[
            pltpu.VMEM((KB, CHUNK), jnp.int32), pltpu.VMEM((KB * CHUNK, D), _F32),
            pltpu.SemaphoreType.DMA,
        ],
    )
    def k(table_h, ii_h, oi_h, iv1, rv1, sem):
        c = lax.axis_index("c")
        s = lax.axis_index("s")
        wid = s * NC + c
        c0, nch = _worker_range(wid)

        def batch(b, carry):
            cb = c0 + b * KB
            pltpu.sync_copy(ii_h.at[pl.ds(cb, KB)], iv1)
            cps = [pltpu.async_copy(table_h.at[iv1.at[kk]],
                                    rv1.at[pl.ds(kk * CHUNK, CHUNK)], sem)
                   for kk in range(KB)]
            for cp in cps:
                cp.wait()
            pltpu.sync_copy(rv1, oi_h.at[pl.ds(cb * CHUNK, KB * CHUNK)])
            return carry

        lax.fori_loop(0, NB, batch, 0)

        def tailb(t, carry):
            r = c0 + NB * KB + t
            pltpu.sync_copy(ii_h.at[r], iv1.at[0])
            pltpu.async_copy(table_h.at[iv1.at[0]],
                             rv1.at[pl.ds(0, CHUNK)], sem).wait()
            pltpu.sync_copy(rv1.at[pl.ds(0, CHUNK)],
                            oi_h.at[pl.ds(r * CHUNK, CHUNK)])
            return carry

        lax.fori_loop(0, nch - NB * KB, tailb, 0)

    return k(table, idx_i)


def _sc_scatter_add(vals, idx, zeros):
    @functools.partial(
        pl.kernel,
        out_type=jax.ShapeDtypeStruct((NC, N, D), _F32),
        mesh=_mesh(),
        compiler_params=pltpu.CompilerParams(use_tc_tiling_on_sc=False),
        scratch_types=[
            pltpu.VMEM_SHARED((N, D), _F32),
            pltpu.VMEM((KBS, CHUNK), jnp.int32), pltpu.VMEM((KBS * CHUNK, D), _F32),
            pltpu.SemaphoreType.DMA,
        ],
    )
    def k(vals_h, idx_h, z_h, out_h, acc_sh, iv, rv, sem):
        c = lax.axis_index("c")
        s = lax.axis_index("s")
        wid = s * NC + c
        row0 = s * ROWS_PER_TILE
        pltpu.sync_copy(z_h, acc_sh.at[pl.ds(row0, ROWS_PER_TILE)])
        plsc.subcore_barrier()
        c0, nch = _worker_range(wid)

        def batch(b, carry):
            cb = c0 + b * KBS
            pltpu.sync_copy(idx_h.at[pl.ds(cb, KBS)], iv)
            pltpu.sync_copy(vals_h.at[pl.ds(cb * CHUNK, KBS * CHUNK)], rv)
            cps = [pltpu.async_copy(rv.at[pl.ds(kk * CHUNK, CHUNK)],
                                    acc_sh.at[iv.at[kk]], sem, add=True)
                   for kk in range(KBS)]
            for cp in cps:
                cp.wait()
            return carry

        lax.fori_loop(0, NBS, batch, 0)

        def tailb(t, carry):
            r = c0 + NBS * KBS + t
            pltpu.sync_copy(idx_h.at[r], iv.at[0])
            pltpu.sync_copy(vals_h.at[pl.ds(r * CHUNK, CHUNK)],
                            rv.at[pl.ds(0, CHUNK)])
            pltpu.sync_copy(rv.at[pl.ds(0, CHUNK)], acc_sh.at[iv.at[0]], add=True)
            return carry

        lax.fori_loop(0, nch - NBS * KBS, tailb, 0)
        plsc.subcore_barrier()
        pltpu.sync_copy(acc_sh.at[pl.ds(row0, ROWS_PER_TILE)],
                        out_h.at[c, pl.ds(row0, ROWS_PER_TILE)])

    return k(vals, idx, zeros)


def _sc_count(idx, zeros16, ones16):
    @functools.partial(
        pl.kernel,
        out_type=jax.ShapeDtypeStruct((NC, N, 16), _F32),
        mesh=_mesh(),
        compiler_params=pltpu.CompilerParams(use_tc_tiling_on_sc=False),
        scratch_types=[
            pltpu.VMEM_SHARED((N, 16), _F32),
            pltpu.VMEM((KB, CHUNK), jnp.int32), pltpu.VMEM((CHUNK, 16), _F32),
            pltpu.SemaphoreType.DMA,
        ],
    )
    def k(idx_h, z_h, ones_h, out_h, acc_sh, iv, ov, sem):
        c = lax.axis_index("c")
        s = lax.axis_index("s")
        wid = s * NC + c
        row0 = s * ROWS_PER_TILE
        pltpu.sync_copy(z_h, acc_sh.at[pl.ds(row0, ROWS_PER_TILE)])
        pltpu.sync_copy(ones_h, ov)
        plsc.subcore_barrier()
        c0, nch = _worker_range(wid)

        def batch(b, carry):
            cb = c0 + b * KB
            pltpu.sync_copy(idx_h.at[pl.ds(cb, KB)], iv)
            cps = [pltpu.async_copy(ov, acc_sh.at[iv.at[kk]], sem, add=True)
                   for kk in range(KB)]
            for cp in cps:
                cp.wait()
            return carry

        lax.fori_loop(0, NB, batch, 0)

        def tailb(t, carry):
            r = c0 + NB * KB + t
            pltpu.sync_copy(idx_h.at[r], iv.at[0])
            pltpu.sync_copy(ov, acc_sh.at[iv.at[0]], add=True)
            return carry

        lax.fori_loop(0, nch - NB * KB, tailb, 0)
        plsc.subcore_barrier()
        pltpu.sync_copy(acc_sh.at[pl.ds(row0, ROWS_PER_TILE)],
                        out_h.at[c, pl.ds(row0, ROWS_PER_TILE)])

    return k(idx, zeros16, ones16)


def _bd4(w):
    return jax.scipy.linalg.block_diag(w, w, w, w)


def _bt4(bvec):
    return jnp.tile(bvec.reshape(1, -1), (1, 4))


def kernel(x, edge_attr, edge_index, W_s_enc_node, b_s_enc_node, W_enc_node,
           b_enc_node, W_e_enc_node, b_e_enc_node, W_s_enc_edge, b_s_enc_edge,
           W_enc_edge, b_enc_edge, W_e_enc_edge, b_e_enc_edge, W_s_node,
           W_node, W_e_node, W_s_edge, b_s_edge, W_edge, b_edge, W_e_edge,
           b_e_edge):
    idx_i = edge_index[0].reshape(NCHUNKS, CHUNK)
    idx_j = edge_index[1].reshape(NCHUNKS, CHUNK)
    ea_t = jnp.transpose(edge_attr)

    zeros_d = jnp.zeros((ROWS_PER_TILE, D), _F32)
    zeros16 = jnp.zeros((ROWS_PER_TILE, 16), _F32)
    ones16 = jnp.ones((CHUNK, 16), _F32)

    r2 = lambda v: v.reshape(1, -1)

    h_node = _node_encoder(x, W_s_enc_node, r2(b_s_enc_node), W_enc_node,
                           r2(b_enc_node), W_e_enc_node, r2(b_e_enc_node))
    cnt = _sc_count(idx_j, zeros16, ones16)
    inv = _inv_counts(cnt)

    gi = _sc_gather1(h_node, idx_i).reshape(E4, 128)
    h_edge, m = _edge_encoder(ea_t, gi, W_s_enc_edge, r2(b_s_enc_edge),
                              W_enc_edge, r2(b_enc_edge),
                              W_e_enc_edge, r2(b_e_enc_edge),
                              _bd4(W_s_node[:D]), _bd4(W_s_node[D:]))
    acc = _sc_scatter_add(m.reshape(E, D), idx_j, zeros_d)
    h_node = _node_finalize(acc, inv)

    for w_eu, b_eu, w_msg in ((W_s_edge, b_s_edge, W_node),
                              (W_edge, b_edge, W_node),
                              (W_edge, b_edge, W_e_node)):
        gi, gj = _sc_gather2(h_node, idx_i, idx_j)
        gi = gi.reshape(E4, 128)
        gj = gj.reshape(E4, 128)
        h_edge, m = _fused_edge(h_edge, gi, gj, _bd4(w_eu[:D]),
                                _bd4(w_eu[D:2 * D]), _bd4(w_eu[2 * D:]),
                                _bt4(b_eu), _bd4(w_msg[:D]), _bd4(w_msg[D:]))
        acc = _sc_scatter_add(m.reshape(E, D), idx_j, zeros_d)
        h_node = _node_finalize(acc, inv)

    gi, gj = _sc_gather2(h_node, idx_i, idx_j)
    gi = gi.reshape(E4, 128)
    gj = gj.reshape(E4, 128)
    res = _final_edge(h_edge, gi, gj, _bd4(W_e_edge[:D]),
                      _bd4(W_e_edge[D:2 * D]), _bd4(W_e_edge[2 * D:]),
                      _bt4(b_e_edge))
    return edge_attr + res.reshape(E, 3)
